# baseline (speedup 1.0000x reference)
import functools

import jax
import jax.numpy as jnp
from jax import lax
from jax.experimental import pallas as pl
from jax.experimental.pallas import tpu as pltpu
from jax.experimental.pallas import tpu_sc as plsc

VOCAB = 100000
SEQ = 2048
BATCH = 4
EMBED = 1024

NC = 2
NS = 16
L = 16
NW = NC * NS

ROWS = BATCH * SEQ
RPW = ROWS // NW
CHUNK = 32
NCHUNK = RPW // CHUNK

_GATHER_DNUMS = lax.GatherDimensionNumbers(
    offset_dims=(), collapsed_slice_dims=(0,), start_index_map=(0,)
)


def _lane_sum(v):
    for sh in (8, 4, 2, 1):
        idx = lax.iota(jnp.int32, L) ^ sh
        v = v + lax.gather(
            v, idx[:, None], _GATHER_DNUMS, (1,),
            mode=lax.GatherScatterMode.PROMISE_IN_BOUNDS,
        )
    return v


_mesh = plsc.VectorSubcoreMesh(
    core_axis_name="c", subcore_axis_name="s", num_cores=NC, num_subcores=NS
)


@functools.partial(
    pl.kernel,
    out_type=jax.ShapeDtypeStruct((ROWS, EMBED), jnp.float32),
    mesh=_mesh,
    compiler_params=pltpu.CompilerParams(needs_layout_passes=False),
    scratch_types=[
        pltpu.VMEM((RPW,), jnp.int32),
        pltpu.VMEM((CHUNK, EMBED), jnp.float32),
        pltpu.VMEM((CHUNK, EMBED), jnp.float32),
        pltpu.VMEM((EMBED,), jnp.float32),
        pltpu.VMEM((EMBED,), jnp.float32),
        pltpu.SemaphoreType.DMA,
    ],
)
def _emb_ln(ids_hbm, table_hbm, pos_hbm, gamma_hbm, beta_hbm, out_hbm,
            idx_v, tok_v, pos_v, g_v, b_v, sem):
    wid = lax.axis_index("s") * NC + lax.axis_index("c")
    base = wid * RPW
    pos_base = lax.rem(base, SEQ)

    pltpu.sync_copy(ids_hbm.at[pl.ds(base, RPW)], idx_v)
    pltpu.sync_copy(gamma_hbm, g_v)
    pltpu.sync_copy(beta_hbm, b_v)

    for c in range(NCHUNK):
        pltpu.async_copy(
            table_hbm.at[idx_v.at[pl.ds(c * CHUNK, CHUNK)]], tok_v, sem
        ).wait()
        pltpu.sync_copy(pos_hbm.at[pl.ds(pos_base + c * CHUNK, CHUNK)], pos_v)

        def row_body(r, _):
            def pass1(j, carry):
                s, q = carry
                sl = pl.ds(j * L, L)
                v = tok_v[r, sl] + pos_v[r, sl]
                tok_v[r, sl] = v
                return (s + v, q + v * v)

            zero = jnp.zeros((L,), jnp.float32)
            s, q = lax.fori_loop(0, EMBED // L, pass1, (zero, zero))
            mean_v = _lane_sum(s) * (1.0 / EMBED)
            vv = _lane_sum(q) * (1.0 / EMBED) - mean_v * mean_v + 1e-5

            y = plsc.bitcast(
                jnp.int32(0x5F3759DF) - (plsc.bitcast(vv, jnp.int32) >> 1),
                jnp.float32,
            )
            for _ in range(3):
                y = y * (1.5 - 0.5 * vv * y * y)

            def pass2(j, carry):
                sl = pl.ds(j * L, L)
                v = tok_v[r, sl]
                tok_v[r, sl] = (v - mean_v) * y * g_v[sl] + b_v[sl]
                return carry

            lax.fori_loop(0, EMBED // L, pass2, 0)
            return 0

        lax.fori_loop(0, CHUNK, row_body, 0)
        pltpu.sync_copy(tok_v, out_hbm.at[pl.ds(base + c * CHUNK, CHUNK)])


def kernel(input_ids, token_table, pos_table, gamma, beta):
    flat_ids = input_ids.reshape(-1).astype(jnp.int32)
    out = _emb_ln(flat_ids, token_table, pos_table, gamma, beta)
    return out.reshape(BATCH, SEQ, EMBED)


# speedup vs baseline: 1.5564x; 1.5564x over previous
import functools

import jax
import jax.numpy as jnp
from jax import lax
from jax.experimental import pallas as pl
from jax.experimental.pallas import tpu as pltpu
from jax.experimental.pallas import tpu_sc as plsc

VOCAB = 100000
SEQ = 2048
BATCH = 4
EMBED = 1024

NC = 2
NS = 16
L = 16
NW = NC * NS

ROWS = BATCH * SEQ
RPW = ROWS // NW
CHUNK = 16
NCHUNK = RPW // CHUNK
NSLICE = EMBED // L

_GATHER_DNUMS = lax.GatherDimensionNumbers(
    offset_dims=(), collapsed_slice_dims=(0,), start_index_map=(0,)
)


def _lane_sum(v):
    for sh in (8, 4, 2, 1):
        idx = lax.iota(jnp.int32, L) ^ sh
        v = v + lax.gather(
            v, idx[:, None], _GATHER_DNUMS, (1,),
            mode=lax.GatherScatterMode.PROMISE_IN_BOUNDS,
        )
    return v


_mesh = plsc.VectorSubcoreMesh(
    core_axis_name="c", subcore_axis_name="s", num_cores=NC, num_subcores=NS
)


@functools.partial(
    pl.kernel,
    out_type=jax.ShapeDtypeStruct((ROWS, EMBED), jnp.float32),
    mesh=_mesh,
    compiler_params=pltpu.CompilerParams(needs_layout_passes=False),
    scratch_types=[
        pltpu.VMEM((RPW,), jnp.int32),
        pltpu.VMEM((CHUNK, EMBED), jnp.float32),
        pltpu.VMEM((CHUNK, EMBED), jnp.float32),
        pltpu.VMEM((CHUNK, EMBED), jnp.float32),
        pltpu.VMEM((CHUNK, EMBED), jnp.float32),
        pltpu.VMEM((EMBED,), jnp.float32),
        pltpu.VMEM((EMBED,), jnp.float32),
        pltpu.SemaphoreType.DMA,
        pltpu.SemaphoreType.DMA,
        pltpu.SemaphoreType.DMA,
        pltpu.SemaphoreType.DMA,
    ],
)
def _emb_ln(ids_hbm, table_hbm, pos_hbm, gamma_hbm, beta_hbm, out_hbm,
            idx_v, tok0, pos0, tok1, pos1, g_v, b_v, ts0, ps0, ts1, ps1):
    wid = lax.axis_index("s") * NC + lax.axis_index("c")
    base = wid * RPW
    pos_base = lax.rem(base, SEQ)

    pltpu.sync_copy(ids_hbm.at[pl.ds(base, RPW)], idx_v)
    pltpu.sync_copy(gamma_hbm, g_v)
    pltpu.sync_copy(beta_hbm, b_v)

    def start_fetch(k, tok_buf, pos_buf, tsem, psem):
        pltpu.async_copy(
            table_hbm.at[idx_v.at[pl.ds(k * CHUNK, CHUNK)]], tok_buf, tsem
        )
        pltpu.async_copy(
            pos_hbm.at[pl.ds(pos_base + k * CHUNK, CHUNK)], pos_buf, psem
        )

    def wait_fetch(k, tok_buf, pos_buf, tsem, psem):
        pltpu.make_async_copy(
            table_hbm.at[idx_v.at[pl.ds(k * CHUNK, CHUNK)]], tok_buf, tsem
        ).wait()
        pltpu.make_async_copy(
            pos_hbm.at[pl.ds(pos_base + k * CHUNK, CHUNK)], pos_buf, psem
        ).wait()

    def process_rows(tok_buf, pos_buf):
        def row_body(r, _):
            zero = jnp.zeros((L,), jnp.float32)
            s = [zero] * 4
            q = [zero] * 4
            vals = []
            for j in range(NSLICE):
                sl = pl.ds(j * L, L)
                v = tok_buf[r, sl] + pos_buf[r, sl]
                tok_buf[r, sl] = v
                s[j % 4] = s[j % 4] + v
                q[j % 4] = q[j % 4] + v * v
            s_tot = (s[0] + s[1]) + (s[2] + s[3])
            q_tot = (q[0] + q[1]) + (q[2] + q[3])
            mean_v = _lane_sum(s_tot) * (1.0 / EMBED)
            var_v = _lane_sum(q_tot) * (1.0 / EMBED) - mean_v * mean_v + 1e-5

            y = plsc.bitcast(
                jnp.int32(0x5F3759DF) - (plsc.bitcast(var_v, jnp.int32) >> 1),
                jnp.float32,
            )
            for _ in range(3):
                y = y * (1.5 - 0.5 * var_v * y * y)
            nb = (0.0 - mean_v) * y
            for j in range(NSLICE):
                sl = pl.ds(j * L, L)
                t = tok_buf[r, sl] * y + nb
                tok_buf[r, sl] = t * g_v[sl] + b_v[sl]
            return 0

        lax.fori_loop(0, CHUNK, row_body, 0)

    def phase(step, ph, tok_buf, pos_buf, tsem, psem):
        k = step * 2 + ph
        wait_fetch(k, tok_buf, pos_buf, tsem, psem)
        process_rows(tok_buf, pos_buf)
        pltpu.sync_copy(tok_buf, out_hbm.at[pl.ds(base + k * CHUNK, CHUNK)])

        @pl.when(step < NCHUNK // 2 - 1)
        def _():
            start_fetch(k + 2, tok_buf, pos_buf, tsem, psem)

    start_fetch(0, tok0, pos0, ts0, ps0)
    start_fetch(1, tok1, pos1, ts1, ps1)

    def pair_body(step, _):
        phase(step, 0, tok0, pos0, ts0, ps0)
        phase(step, 1, tok1, pos1, ts1, ps1)
        return 0

    lax.fori_loop(0, NCHUNK // 2, pair_body, 0)


def kernel(input_ids, token_table, pos_table, gamma, beta):
    flat_ids = input_ids.reshape(-1).astype(jnp.int32)
    out = _emb_ln(flat_ids, token_table, pos_table, gamma, beta)
    return out.reshape(BATCH, SEQ, EMBED)


# speedup vs baseline: 3.2727x; 2.1028x over previous
import functools

import jax
import jax.numpy as jnp
from jax import lax
from jax.experimental import pallas as pl
from jax.experimental.pallas import tpu as pltpu
from jax.experimental.pallas import tpu_sc as plsc

VOCAB = 100000
SEQ = 2048
BATCH = 4
EMBED = 1024

NC = 2
NS = 16
NW = NC * NS

ROWS = BATCH * SEQ
RPW = ROWS // NW
CHUNK = 16
NCHUNK = RPW // CHUNK
NBUF = 4

_mesh = plsc.VectorSubcoreMesh(
    core_axis_name="c", subcore_axis_name="s", num_cores=NC, num_subcores=NS
)


@functools.partial(
    pl.kernel,
    out_type=jax.ShapeDtypeStruct((ROWS, EMBED), jnp.float32),
    mesh=_mesh,
    compiler_params=pltpu.CompilerParams(needs_layout_passes=False),
    scratch_types=[
        pltpu.VMEM((RPW,), jnp.int32),
        pltpu.VMEM((NBUF, CHUNK, EMBED), jnp.float32),
        [pltpu.SemaphoreType.DMA] * NBUF,
    ],
)
def _sc_gather(ids_hbm, table_hbm, out_hbm, idx_v, bufs, sems):
    wid = lax.axis_index("s") * NC + lax.axis_index("c")
    base = wid * RPW
    pltpu.sync_copy(ids_hbm.at[pl.ds(base, RPW)], idx_v)

    def start(k, b):
        pltpu.async_copy(
            table_hbm.at[idx_v.at[pl.ds(k * CHUNK, CHUNK)]], bufs.at[b], sems[b]
        )

    def wait(k, b):
        pltpu.make_async_copy(
            table_hbm.at[idx_v.at[pl.ds(k * CHUNK, CHUNK)]], bufs.at[b], sems[b]
        ).wait()

    for b in range(NBUF):
        start(b, b)

    def ring_body(step, _):
        for b in range(NBUF):
            k = step * NBUF + b
            wait(k, b)
            pltpu.sync_copy(bufs.at[b], out_hbm.at[pl.ds(base + k * CHUNK, CHUNK)])

            @pl.when(step < NCHUNK // NBUF - 1)
            def _():
                start(k + NBUF, b)

        return 0

    lax.fori_loop(0, NCHUNK // NBUF, ring_body, 0)


BM = 256


def _tc_ln_body(emb_ref, pos_ref, g_ref, b_ref, out_ref):
    e = emb_ref[...] + pos_ref[...]
    mean = jnp.mean(e, axis=-1, keepdims=True)
    var = jnp.mean((e - mean) ** 2, axis=-1, keepdims=True)
    out_ref[...] = (e - mean) * lax.rsqrt(var + 1e-5) * g_ref[...] + b_ref[...]


_tc_ln = pl.pallas_call(
    _tc_ln_body,
    grid=(ROWS // BM,),
    in_specs=[
        pl.BlockSpec((BM, EMBED), lambda i: (i, 0)),
        pl.BlockSpec((BM, EMBED), lambda i: (i % (SEQ // BM), 0)),
        pl.BlockSpec((1, EMBED), lambda i: (0, 0)),
        pl.BlockSpec((1, EMBED), lambda i: (0, 0)),
    ],
    out_specs=pl.BlockSpec((BM, EMBED), lambda i: (i, 0)),
    out_shape=jax.ShapeDtypeStruct((ROWS, EMBED), jnp.float32),
)


def kernel(input_ids, token_table, pos_table, gamma, beta):
    flat_ids = input_ids.reshape(-1).astype(jnp.int32)
    emb = _sc_gather(flat_ids, token_table)
    out = _tc_ln(emb, pos_table, gamma.reshape(1, EMBED), beta.reshape(1, EMBED))
    return out.reshape(BATCH, SEQ, EMBED)


# speedup vs baseline: 3.3751x; 1.0313x over previous
import functools

import jax
import jax.numpy as jnp
from jax import lax
from jax.experimental import pallas as pl
from jax.experimental.pallas import tpu as pltpu
from jax.experimental.pallas import tpu_sc as plsc

VOCAB = 100000
SEQ = 2048
BATCH = 4
EMBED = 1024

NC = 2
NS = 16
NW = NC * NS

ROWS = BATCH * SEQ
RPW = ROWS // NW
CHUNK = 16
NCHUNK = RPW // CHUNK
NBUF = 4

_mesh = plsc.VectorSubcoreMesh(
    core_axis_name="c", subcore_axis_name="s", num_cores=NC, num_subcores=NS
)


@functools.partial(
    pl.kernel,
    out_type=jax.ShapeDtypeStruct((ROWS, EMBED), jnp.float32),
    mesh=_mesh,
    compiler_params=pltpu.CompilerParams(needs_layout_passes=False),
    scratch_types=[
        pltpu.VMEM((RPW,), jnp.int32),
        pltpu.VMEM((NBUF, CHUNK, EMBED), jnp.float32),
        [pltpu.SemaphoreType.DMA] * NBUF,
    ],
)
def _sc_gather(ids_hbm, table_hbm, out_hbm, idx_v, bufs, sems):
    wid = lax.axis_index("s") * NC + lax.axis_index("c")
    base = wid * RPW
    pltpu.sync_copy(ids_hbm.at[pl.ds(base, RPW)], idx_v)

    def start(k, b):
        pltpu.async_copy(
            table_hbm.at[idx_v.at[pl.ds(k * CHUNK, CHUNK)]], bufs.at[b], sems[b]
        )

    def wait(k, b):
        pltpu.make_async_copy(
            table_hbm.at[idx_v.at[pl.ds(k * CHUNK, CHUNK)]], bufs.at[b], sems[b]
        ).wait()

    for b in range(NBUF):
        start(b, b)

    def ring_body(step, _):
        for b in range(NBUF):
            k = step * NBUF + b
            wait(k, b)
            pltpu.sync_copy(bufs.at[b], out_hbm.at[pl.ds(base + k * CHUNK, CHUNK)])

            @pl.when(step < NCHUNK // NBUF - 1)
            def _():
                start(k + NBUF, b)

        return 0

    lax.fori_loop(0, NCHUNK // NBUF, ring_body, 0)


BM = 256


def _tc_ln_body(emb_ref, pos_ref, g_ref, b_ref, out_ref):
    e = emb_ref[...] + pos_ref[...]
    mean = jnp.mean(e, axis=-1, keepdims=True)
    var = jnp.mean((e - mean) ** 2, axis=-1, keepdims=True)
    out_ref[...] = (e - mean) * lax.rsqrt(var + 1e-5) * g_ref[...] + b_ref[...]


_tc_ln = pl.pallas_call(
    _tc_ln_body,
    grid=(SEQ // BM, BATCH),
    in_specs=[
        pl.BlockSpec((BM, EMBED), lambda i, j: (j * (SEQ // BM) + i, 0)),
        pl.BlockSpec((BM, EMBED), lambda i, j: (i, 0)),
        pl.BlockSpec((1, EMBED), lambda i, j: (0, 0)),
        pl.BlockSpec((1, EMBED), lambda i, j: (0, 0)),
    ],
    out_specs=pl.BlockSpec((BM, EMBED), lambda i, j: (j * (SEQ // BM) + i, 0)),
    out_shape=jax.ShapeDtypeStruct((ROWS, EMBED), jnp.float32),
)


def kernel(input_ids, token_table, pos_table, gamma, beta):
    flat_ids = input_ids.reshape(-1).astype(jnp.int32)
    emb = _sc_gather(flat_ids, token_table)
    out = _tc_ln(emb, pos_table, gamma.reshape(1, EMBED), beta.reshape(1, EMBED))
    return out.reshape(BATCH, SEQ, EMBED)


# speedup vs baseline: 3.7123x; 1.0999x over previous
import functools

import jax
import jax.numpy as jnp
from jax import lax
from jax.experimental import pallas as pl
from jax.experimental.pallas import tpu as pltpu
from jax.experimental.pallas import tpu_sc as plsc

VOCAB = 100000
SEQ = 2048
BATCH = 4
EMBED = 1024

NC = 2
NS = 16
NW = NC * NS

ROWS = BATCH * SEQ
RPW = ROWS // NW
CHUNK = 16
NCHUNK = RPW // CHUNK
NBUF = 4

_mesh = plsc.VectorSubcoreMesh(
    core_axis_name="c", subcore_axis_name="s", num_cores=NC, num_subcores=NS
)


@functools.partial(
    pl.kernel,
    out_type=jax.ShapeDtypeStruct((ROWS, EMBED), jnp.float32),
    mesh=_mesh,
    compiler_params=pltpu.CompilerParams(needs_layout_passes=False),
    scratch_types=[
        pltpu.VMEM((RPW,), jnp.int32),
        pltpu.VMEM((NBUF, CHUNK, EMBED), jnp.float32),
        [pltpu.SemaphoreType.DMA] * NBUF,
    ],
)
def _sc_gather(ids_hbm, table_hbm, out_hbm, idx_v, bufs, sems):
    wid = lax.axis_index("s") * NC + lax.axis_index("c")
    base = wid * RPW
    pltpu.sync_copy(ids_hbm.at[pl.ds(base, RPW)], idx_v)

    def start(k, b):
        pltpu.async_copy(
            table_hbm.at[idx_v.at[pl.ds(k * CHUNK, CHUNK)]], bufs.at[b], sems[b]
        )

    def wait(k, b):
        pltpu.make_async_copy(
            table_hbm.at[idx_v.at[pl.ds(k * CHUNK, CHUNK)]], bufs.at[b], sems[b]
        ).wait()

    for b in range(NBUF):
        start(b, b)

    def ring_body(step, _):
        for b in range(NBUF):
            k = step * NBUF + b
            wait(k, b)
            pltpu.sync_copy(bufs.at[b], out_hbm.at[pl.ds(base + k * CHUNK, CHUNK)])

            @pl.when(step < NCHUNK // NBUF - 1)
            def _():
                start(k + NBUF, b)

        return 0

    lax.fori_loop(0, NCHUNK // NBUF, ring_body, 0)


BM = 512


def _tc_ln_body(emb_ref, pos_ref, g_ref, b_ref, out_ref):
    e = emb_ref[...] + pos_ref[...]
    mean = jnp.mean(e, axis=-1, keepdims=True)
    var = jnp.mean(e * e, axis=-1, keepdims=True) - mean * mean
    out_ref[...] = (e - mean) * lax.rsqrt(var + 1e-5) * g_ref[...] + b_ref[...]


_tc_ln = pl.pallas_call(
    _tc_ln_body,
    grid=(SEQ // BM, BATCH),
    in_specs=[
        pl.BlockSpec((BM, EMBED), lambda i, j: (j * (SEQ // BM) + i, 0)),
        pl.BlockSpec((BM, EMBED), lambda i, j: (i, 0)),
        pl.BlockSpec((1, EMBED), lambda i, j: (0, 0)),
        pl.BlockSpec((1, EMBED), lambda i, j: (0, 0)),
    ],
    out_specs=pl.BlockSpec((BM, EMBED), lambda i, j: (j * (SEQ // BM) + i, 0)),
    out_shape=jax.ShapeDtypeStruct((ROWS, EMBED), jnp.float32),
)


def kernel(input_ids, token_table, pos_table, gamma, beta):
    flat_ids = input_ids.reshape(-1).astype(jnp.int32)
    emb = _sc_gather(flat_ids, token_table)
    out = _tc_ln(emb, pos_table, gamma.reshape(1, EMBED), beta.reshape(1, EMBED))
    return out.reshape(BATCH, SEQ, EMBED)
